# Initial kernel scaffold; baseline (speedup 1.0000x reference)
import functools

import jax
import jax.numpy as jnp
from jax import lax
from jax.experimental import pallas as pl
from jax.experimental.pallas import tpu as pltpu
from jax.experimental.pallas import tpu_sc as plsc

N = 10000
D = 128
E = 320000

NC = 2
NS = 16
NW = NC * NS
EPW = E // NW
CHUNK = 80
NCHUNK = EPW // CHUNK
RPS = N // NS

_sc_mesh = plsc.VectorSubcoreMesh(
    core_axis_name="c", subcore_axis_name="s", num_cores=NC, num_subcores=NS
)


@functools.partial(
    pl.kernel,
    out_type=jax.ShapeDtypeStruct((NC, N, D), jnp.float32),
    mesh=_sc_mesh,
    scratch_types=[
        pltpu.VMEM((NCHUNK, CHUNK), jnp.int32),
        pltpu.VMEM((NCHUNK, CHUNK), jnp.int32),
        pltpu.VMEM((CHUNK, D), jnp.float32),
        pltpu.VMEM((CHUNK, D), jnp.float32),
        pltpu.VMEM_SHARED((N, D), jnp.float32),
        pltpu.SemaphoreType.DMA,
        pltpu.SemaphoreType.DMA,
    ],
)
def _sc_aggregate(table_hbm, src_hbm, dst_hbm, zeros_hbm, out_hbm,
                  src_v, dst_v, buf_a, buf_b, agg_sh, sem_a, sem_b):
    c = lax.axis_index("c")
    s = lax.axis_index("s")
    wid = s * NC + c

    pltpu.sync_copy(zeros_hbm, agg_sh.at[pl.ds(s * RPS, RPS)])
    pltpu.sync_copy(src_hbm.at[wid], src_v)
    pltpu.sync_copy(dst_hbm.at[wid], dst_v)
    plsc.subcore_barrier()

    cp0 = pltpu.async_copy(table_hbm.at[src_v.at[0]], buf_a, sem_a)

    def body(i, _):
        j = i * 2
        cp_b = pltpu.async_copy(table_hbm.at[src_v.at[j + 1]], buf_b, sem_b)
        pltpu.async_copy(table_hbm.at[src_v.at[j]], buf_a, sem_a).wait()
        pltpu.sync_copy(buf_a, agg_sh.at[dst_v.at[j]], add=True)
        pltpu.async_copy(table_hbm.at[src_v.at[j + 2]], buf_a, sem_a)
        cp_b.wait()
        pltpu.sync_copy(buf_b, agg_sh.at[dst_v.at[j + 1]], add=True)
        return 0

    lax.fori_loop(0, (NCHUNK - 1) // 2, body, 0)
    cp0.wait()
    pltpu.sync_copy(buf_a, agg_sh.at[dst_v.at[NCHUNK - 1]], add=True)

    plsc.subcore_barrier()
    pltpu.sync_copy(agg_sh.at[pl.ds(s * RPS, RPS)],
                    out_hbm.at[c, pl.ds(s * RPS, RPS)])


def _mm_relu_body(x_ref, a_ref, w_ref, b_ref, o_ref):
    xa = x_ref[...] + a_ref[0] + a_ref[1]
    h = jnp.dot(xa, w_ref[...], preferred_element_type=jnp.float32)
    o_ref[...] = jnp.maximum(h + b_ref[...], 0.0)


def _mm_lsm_body(x_ref, a_ref, w_ref, b_ref, o_ref):
    xa = x_ref[...] + a_ref[0] + a_ref[1]
    z = jnp.dot(xa, w_ref[...], preferred_element_type=jnp.float32)
    z = z + b_ref[...]
    m = jnp.max(z, axis=1, keepdims=True)
    lse = jnp.log(jnp.sum(jnp.exp(z - m), axis=1, keepdims=True)) + m
    o_ref[...] = z - lse


ROWS_BLK = 1000


def _tc_layer(body, x, aggp, wt, b):
    return pl.pallas_call(
        body,
        out_shape=jax.ShapeDtypeStruct((N, D), jnp.float32),
        grid=(N // ROWS_BLK,),
        in_specs=[
            pl.BlockSpec((ROWS_BLK, D), lambda i: (i, 0)),
            pl.BlockSpec((NC, ROWS_BLK, D), lambda i: (0, i, 0)),
            pl.BlockSpec((D, D), lambda i: (0, 0)),
            pl.BlockSpec((1, D), lambda i: (0, 0)),
        ],
        out_specs=pl.BlockSpec((ROWS_BLK, D), lambda i: (i, 0)),
    )(x, aggp, wt, b)


def kernel(x, edge_index, W1, b1, W2, b2):
    ei = edge_index.astype(jnp.int32)
    src = ei[0].reshape(NW, NCHUNK, CHUNK)
    dst = ei[1].reshape(NW, NCHUNK, CHUNK)
    zeros = jnp.zeros((RPS, D), dtype=jnp.float32)

    agg1 = _sc_aggregate(x, src, dst, zeros)
    h = _tc_layer(_mm_relu_body, x, agg1, W1.T, b1.reshape(1, D))
    agg2 = _sc_aggregate(h, src, dst, zeros)
    out = _tc_layer(_mm_lsm_body, h, agg2, W2.T, b2.reshape(1, D))
    return out


# speedup vs baseline: 9.3674x; 9.3674x over previous
import functools

import jax
import jax.numpy as jnp
from jax import lax
from jax.experimental import pallas as pl
from jax.experimental.pallas import tpu as pltpu
from jax.experimental.pallas import tpu_sc as plsc

N = 10000
D = 128
E = 320000

NC = 2
NS = 16
NW = NC * NS
EPW = E // NW
CHUNK = 100
NCHUNK = EPW // CHUNK
NPAD = 10240
RPS = NPAD // NS

@functools.cache
def _make_sc_aggregate():
    mesh = plsc.VectorSubcoreMesh(
        core_axis_name="c", subcore_axis_name="s",
        num_cores=NC, num_subcores=NS,
    )
    return pl.kernel(
        _sc_aggregate_body,
        out_type=jax.ShapeDtypeStruct((NC, NPAD, D), jnp.float32),
        mesh=mesh,
        scratch_types=[
            pltpu.VMEM((2, 2, CHUNK), jnp.int32),
            pltpu.VMEM((CHUNK, D), jnp.float32),
            pltpu.VMEM((CHUNK, D), jnp.float32),
            pltpu.VMEM_SHARED((NPAD, D), jnp.float32),
            pltpu.SemaphoreType.DMA,
            pltpu.SemaphoreType.DMA,
            pltpu.SemaphoreType.DMA,
        ],
    )


def _sc_aggregate_body(table_hbm, eidx_hbm, zeros_hbm, out_hbm,
                       ring, buf_a, buf_b, agg_sh, isem, gsem_a, gsem_b):
    c = lax.axis_index("c")
    s = lax.axis_index("s")
    wid = s * NC + c

    def wait_idx(slot):
        pltpu.make_async_copy(eidx_hbm.at[wid, 0], ring.at[slot], isem).wait()

    def wait_gather(buf, gsem):
        pltpu.make_async_copy(table_hbm.at[ring.at[0, 0]], buf, gsem).wait()

    pltpu.sync_copy(zeros_hbm, agg_sh.at[pl.ds(s * RPS, RPS)])

    pltpu.async_copy(eidx_hbm.at[wid, 0], ring.at[0], isem)
    wait_idx(0)
    pltpu.async_copy(eidx_hbm.at[wid, 1], ring.at[1], isem)
    pltpu.async_copy(table_hbm.at[ring.at[0, 0]], buf_a, gsem_a)
    plsc.subcore_barrier()

    def body(i, _):
        j = 2 * i
        wait_idx(1)
        pltpu.async_copy(table_hbm.at[ring.at[1, 0]], buf_b, gsem_b)
        wait_gather(buf_a, gsem_a)
        pltpu.sync_copy(buf_a, agg_sh.at[ring.at[0, 1]], add=True)
        pltpu.async_copy(eidx_hbm.at[wid, j + 2], ring.at[0], isem)

        wait_idx(0)
        pltpu.async_copy(table_hbm.at[ring.at[0, 0]], buf_a, gsem_a)
        wait_gather(buf_b, gsem_b)
        pltpu.sync_copy(buf_b, agg_sh.at[ring.at[1, 1]], add=True)
        pltpu.async_copy(eidx_hbm.at[wid, j + 3], ring.at[1], isem)
        return 0

    lax.fori_loop(0, NCHUNK // 2 - 1, body, 0)

    wait_idx(1)
    pltpu.async_copy(table_hbm.at[ring.at[1, 0]], buf_b, gsem_b)
    wait_gather(buf_a, gsem_a)
    pltpu.sync_copy(buf_a, agg_sh.at[ring.at[0, 1]], add=True)
    wait_gather(buf_b, gsem_b)
    pltpu.sync_copy(buf_b, agg_sh.at[ring.at[1, 1]], add=True)

    plsc.subcore_barrier()
    pltpu.sync_copy(agg_sh.at[pl.ds(s * RPS, RPS)],
                    out_hbm.at[c, pl.ds(s * RPS, RPS)])


def _mm_relu_body(x_ref, a_ref, w_ref, b_ref, o_ref):
    xa = x_ref[...] + a_ref[0] + a_ref[1]
    h = jnp.dot(xa, w_ref[...], preferred_element_type=jnp.float32)
    o_ref[...] = jnp.maximum(h + b_ref[...], 0.0)


def _mm_lsm_body(x_ref, a_ref, w_ref, b_ref, o_ref):
    xa = x_ref[...] + a_ref[0] + a_ref[1]
    z = jnp.dot(xa, w_ref[...], preferred_element_type=jnp.float32)
    z = z + b_ref[...]
    m = jnp.max(z, axis=1, keepdims=True)
    lse = jnp.log(jnp.sum(jnp.exp(z - m), axis=1, keepdims=True)) + m
    o_ref[...] = z - lse


ROWS_BLK = 1000


def _tc_layer(body, x, aggp, wt, b):
    return pl.pallas_call(
        body,
        out_shape=jax.ShapeDtypeStruct((N, D), jnp.float32),
        grid=(N // ROWS_BLK,),
        in_specs=[
            pl.BlockSpec((ROWS_BLK, D), lambda i: (i, 0)),
            pl.BlockSpec((NC, ROWS_BLK, D), lambda i: (0, i, 0)),
            pl.BlockSpec((D, D), lambda i: (0, 0)),
            pl.BlockSpec((1, D), lambda i: (0, 0)),
        ],
        out_specs=pl.BlockSpec((ROWS_BLK, D), lambda i: (i, 0)),
    )(x, aggp, wt, b)


def kernel(x, edge_index, W1, b1, W2, b2):
    ei = edge_index.astype(jnp.int32)
    src = ei[0].reshape(NW, NCHUNK, CHUNK)
    dst = ei[1].reshape(NW, NCHUNK, CHUNK)
    eidx = jnp.stack([src, dst], axis=2)
    zeros = jnp.zeros((RPS, D), dtype=jnp.float32)

    sc_aggregate = _make_sc_aggregate()
    agg1 = sc_aggregate(x, eidx, zeros)
    h = _tc_layer(_mm_relu_body, x, agg1, W1.T, b1.reshape(1, D))
    agg2 = sc_aggregate(h, eidx, zeros)
    out = _tc_layer(_mm_lsm_body, h, agg2, W2.T, b2.reshape(1, D))
    return out
